# baseline (speedup 1.0000x reference)
import functools

import jax
import jax.numpy as jnp
from jax import lax
from jax.experimental import pallas as pl
from jax.experimental.pallas import tpu as pltpu
from jax.experimental.pallas import tpu_sc as plsc

IN_F = 4096
OUT_F = 1024
STRIDE = 4
LANES = 16

NUM_CORES = 2
NUM_SUBCORES = 16
NUM_WORKERS = NUM_CORES * NUM_SUBCORES

BR = 1024
SC_ROWS = 6144
ROWS_PER_CHUNK = 8


def _tc_body(x_ref, o_ref):
    lane = lax.broadcasted_iota(jnp.int32, (BR, 128), 1)
    idx = (lane * STRIDE) % 128
    grp = lane // 32
    for c in range(OUT_F // 128):
        ys = []
        for g in range(STRIDE):
            v = x_ref[:, 512 * c + 128 * g:512 * c + 128 * (g + 1)]
            ys.append(jnp.take_along_axis(v, idx, axis=1))
        y01 = jnp.where(grp == 0, ys[0], ys[1])
        y23 = jnp.where(grp == 2, ys[2], ys[3])
        o_ref[:, 128 * c:128 * (c + 1)] = jnp.where(grp < 2, y01, y23)


def _sc_body(x_hbm, out_hbm, in_v, out_v,
             in_sem0, in_sem1, out_sem0, out_sem1):
    R = x_hbm.shape[0]
    row_base = R - SC_ROWS
    rows_per_worker = SC_ROWS // NUM_WORKERS
    n_chunks = rows_per_worker // ROWS_PER_CHUNK
    n_pairs = n_chunks // 2

    wid = lax.axis_index("c") * NUM_SUBCORES + lax.axis_index("s")
    row0 = row_base + wid * rows_per_worker
    out0 = wid * rows_per_worker

    in_sems = (in_sem0, in_sem1)
    out_sems = (out_sem0, out_sem1)

    iota = lax.iota(jnp.int32, LANES)
    col0 = iota * STRIDE

    def in_copy(g, b):
        base = row0 + g * ROWS_PER_CHUNK
        return pltpu.make_async_copy(
            x_hbm.at[pl.ds(base, ROWS_PER_CHUNK)], in_v.at[b], in_sems[b])

    def out_copy(g, b):
        base = out0 + g * ROWS_PER_CHUNK
        return pltpu.make_async_copy(
            out_v.at[b], out_hbm.at[pl.ds(base, ROWS_PER_CHUNK)], out_sems[b])

    def compute(b):
        def row_body(r, _):
            ridx = jnp.full((LANES,), r, jnp.int32)
            for j in range(OUT_F // LANES):
                col = col0 + (j * LANES * STRIDE)
                vals = plsc.load_gather(in_v.at[b], [ridx, col])
                out_v[b, r, pl.ds(j * LANES, LANES)] = vals
            return 0
        lax.fori_loop(0, ROWS_PER_CHUNK, row_body, 0)

    in_copy(0, 0).start()
    in_copy(1, 1).start()

    def pair_body(p, _):
        for b in range(2):
            g = 2 * p + b
            in_copy(g, b).wait()

            @pl.when(p >= 1)
            def _():
                out_copy(g, b).wait()

            compute(b)
            out_copy(g, b).start()

            @pl.when(p < n_pairs - 1)
            def _():
                in_copy(g + 2, b).start()
        return 0

    lax.fori_loop(0, n_pairs, pair_body, 0)
    out_copy(n_chunks - 2, 0).wait()
    out_copy(n_chunks - 1, 1).wait()


def kernel(input):
    B, S, F = input.shape
    R = B * S
    tc_rows = R - SC_ROWS
    x = input.reshape(R, F)

    out_tc = pl.pallas_call(
        _tc_body,
        grid=(tc_rows // BR,),
        in_specs=[pl.BlockSpec((BR, IN_F), lambda i: (i, 0))],
        out_specs=pl.BlockSpec((BR, OUT_F), lambda i: (i, 0)),
        out_shape=jax.ShapeDtypeStruct((tc_rows, OUT_F), jnp.float32),
    )(x)

    mesh = plsc.VectorSubcoreMesh(
        core_axis_name="c", subcore_axis_name="s",
        num_cores=NUM_CORES, num_subcores=NUM_SUBCORES,
    )
    run_sc = pl.kernel(
        _sc_body,
        out_type=jax.ShapeDtypeStruct((SC_ROWS, OUT_F), jnp.float32),
        mesh=mesh,
        scratch_types=[
            pltpu.VMEM((2, ROWS_PER_CHUNK, IN_F), jnp.float32),
            pltpu.VMEM((2, ROWS_PER_CHUNK, OUT_F), jnp.float32),
            pltpu.SemaphoreType.DMA,
            pltpu.SemaphoreType.DMA,
            pltpu.SemaphoreType.DMA,
            pltpu.SemaphoreType.DMA,
        ],
        compiler_params=pltpu.CompilerParams(
            use_tc_tiling_on_sc=False, needs_layout_passes=False,
        ),
    )
    out_sc = run_sc(x)

    out = jnp.concatenate([out_tc, out_sc], axis=0)
    return out.reshape(B, S, OUT_F)


# speedup vs baseline: 3.3846x; 3.3846x over previous
import jax
import jax.numpy as jnp
from jax import lax
from jax.experimental import pallas as pl
from jax.experimental.pallas import tpu as pltpu

IN_F = 4096
OUT_F = 1024
STRIDE = 4
BR = 1024


def _tc_body(x_ref, o_ref):
    lane = lax.broadcasted_iota(jnp.int32, (BR, 128), 1)
    idx = (lane * STRIDE) % 128
    grp = lane // 32
    for c in range(OUT_F // 128):
        ys = []
        for g in range(STRIDE):
            v = x_ref[:, 512 * c + 128 * g:512 * c + 128 * (g + 1)]
            ys.append(jnp.take_along_axis(v, idx, axis=1))
        y01 = jnp.where(grp == 0, ys[0], ys[1])
        y23 = jnp.where(grp == 2, ys[2], ys[3])
        o_ref[:, 128 * c:128 * (c + 1)] = jnp.where(grp < 2, y01, y23)


def kernel(input):
    B, S, F = input.shape
    R = B * S
    x = input.reshape(R, F)
    out = pl.pallas_call(
        _tc_body,
        grid=(R // BR,),
        in_specs=[pl.BlockSpec((BR, IN_F), lambda i: (i, 0))],
        out_specs=pl.BlockSpec((BR, OUT_F), lambda i: (i, 0)),
        out_shape=jax.ShapeDtypeStruct((R, OUT_F), jnp.float32),
    )(x)
    return out.reshape(B, S, OUT_F)
